# baseline (speedup 1.0000x reference)
import functools

import jax
import jax.numpy as jnp
from jax import lax
from jax.experimental import pallas as pl
from jax.experimental.pallas import tpu as pltpu
from jax.experimental.pallas import tpu_sc as plsc

_NC = 2
_NS = 16
_NW = _NC * _NS
_K = 128


def _pad_edges(src, dst, n, acc_rows, e_pad):
    e = src.shape[0]
    p = e_pad - e
    ar = jnp.arange(p, dtype=jnp.int32)
    pad_src = ar % n
    pad_dst = n + ar % (acc_rows - n)
    src_p = jnp.concatenate([src, pad_src])
    dst_p = jnp.concatenate([dst, pad_dst])
    return src_p, dst_p


def _make_hist(acc_rows, steps, d):
    mesh = plsc.VectorSubcoreMesh(core_axis_name="c", subcore_axis_name="s")
    rows_per_tile = acc_rows // _NS
    zc = rows_per_tile // _K

    @functools.partial(
        pl.kernel,
        out_type=jax.ShapeDtypeStruct((_NC, acc_rows, d), jnp.float32),
        mesh=mesh,
        scratch_types=[
            pltpu.VMEM((steps, _K), jnp.int32),
            pltpu.VMEM((_K, d), jnp.float32),
            pltpu.VMEM_SHARED((acc_rows, d), jnp.float32),
        ],
    )
    def hist(dst_hbm, out, dst_v, ones_v, acc):
        c = lax.axis_index("c")
        s = lax.axis_index("s")
        wid = s * _NC + c
        base = s * rows_per_tile

        @pl.loop(0, _K)
        def _(i):
            for cc in range(d // 16):
                ones_v[i, pl.ds(cc * 16, 16)] = jnp.zeros((16,), jnp.float32)

        for t in range(zc):
            pltpu.sync_copy(ones_v, acc.at[pl.ds(base + t * _K, _K)])

        @pl.loop(0, _K)
        def _(i):
            for cc in range(d // 16):
                ones_v[i, pl.ds(cc * 16, 16)] = jnp.ones((16,), jnp.float32)

        pltpu.sync_copy(dst_hbm.at[wid], dst_v)
        plsc.subcore_barrier()

        @pl.loop(0, steps)
        def _(j):
            pltpu.sync_copy(ones_v, acc.at[dst_v.at[j]], add=True)

        plsc.subcore_barrier()
        pltpu.sync_copy(acc.at[pl.ds(base, rows_per_tile)],
                        out.at[c, pl.ds(base, rows_per_tile)])

    return hist


def _make_scatter(n, acc_rows, steps, d):
    mesh = plsc.VectorSubcoreMesh(core_axis_name="c", subcore_axis_name="s")
    rows_per_tile = acc_rows // _NS
    zc = rows_per_tile // _K

    @functools.partial(
        pl.kernel,
        out_type=jax.ShapeDtypeStruct((_NC, acc_rows, d), jnp.float32),
        mesh=mesh,
        scratch_types=[
            pltpu.VMEM((steps, _K), jnp.int32),
            pltpu.VMEM((steps, _K), jnp.int32),
            pltpu.VMEM((_K, d), jnp.float32),
            pltpu.VMEM_SHARED((acc_rows, d), jnp.float32),
        ],
    )
    def scatter(table, src_hbm, dst_hbm, out, src_v, dst_v, row, acc):
        c = lax.axis_index("c")
        s = lax.axis_index("s")
        wid = s * _NC + c
        base = s * rows_per_tile

        @pl.loop(0, _K)
        def _(i):
            for cc in range(d // 16):
                row[i, pl.ds(cc * 16, 16)] = jnp.zeros((16,), jnp.float32)

        for t in range(zc):
            pltpu.sync_copy(row, acc.at[pl.ds(base + t * _K, _K)])

        pltpu.sync_copy(src_hbm.at[wid], src_v)
        pltpu.sync_copy(dst_hbm.at[wid], dst_v)
        plsc.subcore_barrier()

        @pl.loop(0, steps)
        def _(j):
            pltpu.sync_copy(table.at[src_v.at[j]], row)
            pltpu.sync_copy(row, acc.at[dst_v.at[j]], add=True)

        plsc.subcore_barrier()
        pltpu.sync_copy(acc.at[pl.ds(base, rows_per_tile)],
                        out.at[c, pl.ds(base, rows_per_tile)])

    return scatter


def _dis(d0, d1):
    return lax.rsqrt(d0[:, 0:1] + d1[:, 0:1] + 1.0)


def _mm1_body(x_ref, w_ref, d0_ref, d1_ref, o_ref):
    dis = _dis(d0_ref[...], d1_ref[...])
    h = jnp.dot(x_ref[...], w_ref[...], preferred_element_type=jnp.float32)
    o_ref[...] = h * dis


def _mm2_body(p0_ref, p1_ref, hs_ref, d0_ref, d1_ref, w_ref, b_ref, o_ref):
    dis = _dis(d0_ref[...], d1_ref[...])
    z = dis * (p0_ref[...] + p1_ref[...] + hs_ref[...]) + b_ref[...]
    z = jnp.maximum(z, 0.0)
    h = jnp.dot(z, w_ref[...], preferred_element_type=jnp.float32)
    o_ref[...] = h * dis


def _fin_body(p0_ref, p1_ref, hs_ref, d0_ref, d1_ref, b_ref,
              fw1_ref, fb1_ref, fw2_ref, fb2_ref, o_ref, acc_ref,
              *, n_total, num_blocks):
    i = pl.program_id(0)
    dis = _dis(d0_ref[...], d1_ref[...])
    z = dis * (p0_ref[...] + p1_ref[...] + hs_ref[...]) + b_ref[...]
    z = jnp.maximum(z, 0.0)
    part = jnp.sum(z, axis=0, keepdims=True)

    @pl.when(i == 0)
    def _():
        acc_ref[...] = part

    @pl.when(i > 0)
    def _():
        acc_ref[...] = acc_ref[...] + part

    @pl.when(i == num_blocks - 1)
    def _():
        g = acc_ref[...] * (1.0 / n_total)
        a = jnp.maximum(
            jnp.dot(g, fw1_ref[...], preferred_element_type=jnp.float32)
            + fb1_ref[...], 0.0)
        o_ref[...] = (jnp.dot(a, fw2_ref[...],
                              preferred_element_type=jnp.float32)
                      + fb2_ref[...])


def kernel(x, edge_index, W1, b1, W2, b2, fW1, fb1, fW2, fb2):
    n, dim = x.shape
    h = W1.shape[1]
    e = edge_index.shape[1]
    acc_rows = ((n + 16 * _K - 1) // (16 * _K)) * (16 * _K)
    unit = _NW * _K
    e_pad = -(-e // unit) * unit
    hsteps = e_pad // (_NW * _K)

    src, dst = edge_index[0], edge_index[1]
    src_p, dst_p = _pad_edges(src, dst, n, acc_rows, e_pad)
    dst_h = dst_p.reshape(_NW, hsteps, _K)
    src_s = src_p.reshape(_NW, hsteps, _K)
    dst_s = dst_h

    dh = 128
    degp = _make_hist(acc_rows, hsteps, dh)(dst_h)
    d0, d1 = degp[0], degp[1]

    nb = 10
    blk = n // nb
    row_spec = lambda last: pl.BlockSpec((blk, last), lambda i: (i, 0))
    full_spec = lambda a, b: pl.BlockSpec((a, b), lambda i: (0, 0))

    hs1 = pl.pallas_call(
        _mm1_body,
        grid=(nb,),
        in_specs=[row_spec(dim), full_spec(dim, h), row_spec(dh),
                  row_spec(dh)],
        out_specs=row_spec(h),
        out_shape=jax.ShapeDtypeStruct((n, h), jnp.float32),
    )(x, W1, d0, d1)

    sc_scatter = _make_scatter(n, acc_rows, hsteps, h)
    p = sc_scatter(hs1, src_s, dst_s)

    hs2 = pl.pallas_call(
        _mm2_body,
        grid=(nb,),
        in_specs=[row_spec(h), row_spec(h), row_spec(h), row_spec(dh),
                  row_spec(dh), full_spec(h, h), full_spec(1, h)],
        out_specs=row_spec(h),
        out_shape=jax.ShapeDtypeStruct((n, h), jnp.float32),
    )(p[0], p[1], hs1, d0, d1, W2, b1.reshape(1, h))

    p2 = sc_scatter(hs2, src_s, dst_s)

    c = fW2.shape[1]
    hh = fW1.shape[1]
    out = pl.pallas_call(
        functools.partial(_fin_body, n_total=n, num_blocks=nb),
        grid=(nb,),
        in_specs=[row_spec(h), row_spec(h), row_spec(h), row_spec(dh),
                  row_spec(dh), full_spec(1, h), full_spec(h, hh),
                  full_spec(1, hh), full_spec(hh, c), full_spec(1, c)],
        out_specs=pl.BlockSpec((1, c), lambda i: (0, 0)),
        out_shape=jax.ShapeDtypeStruct((1, c), jnp.float32),
        scratch_shapes=[pltpu.VMEM((1, h), jnp.float32)],
    )(p2[0], p2[1], hs2, d0, d1, b2.reshape(1, h), fW1,
      fb1.reshape(1, hh), fW2, fb2.reshape(1, c))

    return out


# speedup vs baseline: 1.3635x; 1.3635x over previous
import functools

import jax
import jax.numpy as jnp
from jax import lax
from jax.experimental import pallas as pl
from jax.experimental.pallas import tpu as pltpu
from jax.experimental.pallas import tpu_sc as plsc

_NC = 2
_NS = 16
_NW = _NC * _NS
_K = 128


def _pad_edges(src, dst, n, acc_rows, e_pad):
    e = src.shape[0]
    p = e_pad - e
    ar = jnp.arange(p, dtype=jnp.int32)
    pad_src = ar % n
    pad_dst = n + ar % (acc_rows - n)
    src_p = jnp.concatenate([src, pad_src])
    dst_p = jnp.concatenate([dst, pad_dst])
    return src_p, dst_p


def _make_hist(acc_rows, steps, d):
    mesh = plsc.VectorSubcoreMesh(core_axis_name="c", subcore_axis_name="s")
    rows_per_tile = acc_rows // _NS
    zc = rows_per_tile // _K

    @functools.partial(
        pl.kernel,
        out_type=jax.ShapeDtypeStruct((_NC, acc_rows, d), jnp.float32),
        mesh=mesh,
        scratch_types=[
            pltpu.VMEM((steps, _K), jnp.int32),
            pltpu.VMEM((_K, d), jnp.float32),
            pltpu.VMEM_SHARED((acc_rows, d), jnp.float32),
        ],
    )
    def hist(dst_hbm, out, dst_v, ones_v, acc):
        c = lax.axis_index("c")
        s = lax.axis_index("s")
        wid = s * _NC + c
        base = s * rows_per_tile

        @pl.loop(0, _K)
        def _(i):
            for cc in range(d // 16):
                ones_v[i, pl.ds(cc * 16, 16)] = jnp.zeros((16,), jnp.float32)

        for t in range(zc):
            pltpu.sync_copy(ones_v, acc.at[pl.ds(base + t * _K, _K)])

        @pl.loop(0, _K)
        def _(i):
            for cc in range(d // 16):
                ones_v[i, pl.ds(cc * 16, 16)] = jnp.ones((16,), jnp.float32)

        pltpu.sync_copy(dst_hbm.at[wid], dst_v)
        plsc.subcore_barrier()

        @pl.loop(0, steps)
        def _(j):
            pltpu.sync_copy(ones_v, acc.at[dst_v.at[j]], add=True)

        plsc.subcore_barrier()
        pltpu.sync_copy(acc.at[pl.ds(base, rows_per_tile)],
                        out.at[c, pl.ds(base, rows_per_tile)])

    return hist


_NCH = 4


def _make_scatter(n, acc_rows, steps, d):
    mesh = plsc.VectorSubcoreMesh(core_axis_name="c", subcore_axis_name="s")
    rows_per_tile = acc_rows // _NS
    zc = rows_per_tile // _K
    cs = steps // _NCH
    gl = (cs - 2) // 2

    @functools.partial(
        pl.kernel,
        out_type=jax.ShapeDtypeStruct((_NC, acc_rows, d), jnp.float32),
        mesh=mesh,
        scratch_types=[
            [pltpu.VMEM((cs, _K), jnp.int32)] * 2,
            pltpu.VMEM((steps, _K), jnp.int32),
            [pltpu.VMEM((_K, d), jnp.float32)] * 2,
            [pltpu.SemaphoreType.DMA] * 2,
            [pltpu.SemaphoreType.DMA] * 2,
            pltpu.VMEM_SHARED((acc_rows, d), jnp.float32),
        ],
    )
    def scatter(table, src_hbm, dst_hbm, out, src_c, dst_v, rows, gsems,
                csems, acc):
        c = lax.axis_index("c")
        s = lax.axis_index("s")
        wid = s * _NC + c
        base = s * rows_per_tile

        @pl.loop(0, _K)
        def _(i):
            for cc in range(d // 16):
                rows[0][i, pl.ds(cc * 16, 16)] = jnp.zeros((16,), jnp.float32)

        for t in range(zc):
            pltpu.sync_copy(rows[0], acc.at[pl.ds(base + t * _K, _K)])

        pltpu.sync_copy(dst_hbm.at[wid], dst_v)
        pltpu.async_copy(src_hbm.at[wid, 0], src_c[0], csems[0])
        pltpu.async_copy(src_hbm.at[wid, 1], src_c[1], csems[1])
        pltpu.make_async_copy(src_hbm.at[wid, 0], src_c[0], csems[0]).wait()
        pltpu.async_copy(table.at[src_c[0].at[0]], rows[0], gsems[0])
        pltpu.async_copy(table.at[src_c[0].at[1]], rows[1], gsems[1])
        plsc.subcore_barrier()

        for ch in range(_NCH):
            cb = ch % 2

            @pl.loop(0, gl)
            def _(g):
                for b in range(2):
                    t = g * 2 + b
                    j = ch * cs + t
                    pltpu.make_async_copy(table.at[src_c[cb].at[t]],
                                          rows[b], gsems[b]).wait()
                    pltpu.sync_copy(rows[b], acc.at[dst_v.at[j]], add=True)
                    pltpu.async_copy(table.at[src_c[cb].at[t + 2]],
                                     rows[b], gsems[b])

            if ch + 1 < _NCH:
                pltpu.make_async_copy(src_hbm.at[wid, ch + 1],
                                      src_c[1 - cb], csems[1 - cb]).wait()
            for b in range(2):
                t = cs - 2 + b
                j = ch * cs + t
                pltpu.make_async_copy(table.at[src_c[cb].at[t]],
                                      rows[b], gsems[b]).wait()
                pltpu.sync_copy(rows[b], acc.at[dst_v.at[j]], add=True)
                if ch + 1 < _NCH:
                    pltpu.async_copy(table.at[src_c[1 - cb].at[b]],
                                     rows[b], gsems[b])
            if ch + 2 < _NCH:
                pltpu.async_copy(src_hbm.at[wid, ch + 2], src_c[cb],
                                 csems[cb])

        plsc.subcore_barrier()
        pltpu.sync_copy(acc.at[pl.ds(base, rows_per_tile)],
                        out.at[c, pl.ds(base, rows_per_tile)])

    return scatter


def _dis(d0, d1):
    return lax.rsqrt(d0[:, 0:1] + d1[:, 0:1] + 1.0)


def _mm1_body(x_ref, w_ref, d0_ref, d1_ref, o_ref):
    dis = _dis(d0_ref[...], d1_ref[...])
    h = jnp.dot(x_ref[...], w_ref[...], preferred_element_type=jnp.float32)
    o_ref[...] = h * dis


def _mm2_body(p0_ref, p1_ref, hs_ref, d0_ref, d1_ref, w_ref, b_ref, o_ref):
    dis = _dis(d0_ref[...], d1_ref[...])
    z = dis * (p0_ref[...] + p1_ref[...] + hs_ref[...]) + b_ref[...]
    z = jnp.maximum(z, 0.0)
    h = jnp.dot(z, w_ref[...], preferred_element_type=jnp.float32)
    o_ref[...] = h * dis


def _fin_body(p0_ref, p1_ref, hs_ref, d0_ref, d1_ref, b_ref,
              fw1_ref, fb1_ref, fw2_ref, fb2_ref, o_ref, acc_ref,
              *, n_total, num_blocks):
    i = pl.program_id(0)
    dis = _dis(d0_ref[...], d1_ref[...])
    z = dis * (p0_ref[...] + p1_ref[...] + hs_ref[...]) + b_ref[...]
    z = jnp.maximum(z, 0.0)
    part = jnp.sum(z, axis=0, keepdims=True)

    @pl.when(i == 0)
    def _():
        acc_ref[...] = part

    @pl.when(i > 0)
    def _():
        acc_ref[...] = acc_ref[...] + part

    @pl.when(i == num_blocks - 1)
    def _():
        g = acc_ref[...] * (1.0 / n_total)
        a = jnp.maximum(
            jnp.dot(g, fw1_ref[...], preferred_element_type=jnp.float32)
            + fb1_ref[...], 0.0)
        o_ref[...] = (jnp.dot(a, fw2_ref[...],
                              preferred_element_type=jnp.float32)
                      + fb2_ref[...])


def kernel(x, edge_index, W1, b1, W2, b2, fW1, fb1, fW2, fb2):
    n, dim = x.shape
    h = W1.shape[1]
    e = edge_index.shape[1]
    acc_rows = ((n + 16 * _K - 1) // (16 * _K)) * (16 * _K)
    unit = _NW * _K * _NCH
    e_pad = -(-e // unit) * unit
    hsteps = e_pad // (_NW * _K)

    src, dst = edge_index[0], edge_index[1]
    src_p, dst_p = _pad_edges(src, dst, n, acc_rows, e_pad)
    dst_h = dst_p.reshape(_NW, hsteps, _K)
    src_s = src_p.reshape(_NW, _NCH, hsteps // _NCH, _K)
    dst_s = dst_h

    dh = 128
    degp = _make_hist(acc_rows, hsteps, dh)(dst_h)
    d0, d1 = degp[0], degp[1]

    nb = 10
    blk = n // nb
    row_spec = lambda last: pl.BlockSpec((blk, last), lambda i: (i, 0))
    full_spec = lambda a, b: pl.BlockSpec((a, b), lambda i: (0, 0))

    hs1 = pl.pallas_call(
        _mm1_body,
        grid=(nb,),
        in_specs=[row_spec(dim), full_spec(dim, h), row_spec(dh),
                  row_spec(dh)],
        out_specs=row_spec(h),
        out_shape=jax.ShapeDtypeStruct((n, h), jnp.float32),
    )(x, W1, d0, d1)

    sc_scatter = _make_scatter(n, acc_rows, hsteps, h)
    p = sc_scatter(hs1, src_s, dst_s)

    hs2 = pl.pallas_call(
        _mm2_body,
        grid=(nb,),
        in_specs=[row_spec(h), row_spec(h), row_spec(h), row_spec(dh),
                  row_spec(dh), full_spec(h, h), full_spec(1, h)],
        out_specs=row_spec(h),
        out_shape=jax.ShapeDtypeStruct((n, h), jnp.float32),
    )(p[0], p[1], hs1, d0, d1, W2, b1.reshape(1, h))

    p2 = sc_scatter(hs2, src_s, dst_s)

    c = fW2.shape[1]
    hh = fW1.shape[1]
    out = pl.pallas_call(
        functools.partial(_fin_body, n_total=n, num_blocks=nb),
        grid=(nb,),
        in_specs=[row_spec(h), row_spec(h), row_spec(h), row_spec(dh),
                  row_spec(dh), full_spec(1, h), full_spec(h, hh),
                  full_spec(1, hh), full_spec(hh, c), full_spec(1, c)],
        out_specs=pl.BlockSpec((1, c), lambda i: (0, 0)),
        out_shape=jax.ShapeDtypeStruct((1, c), jnp.float32),
        scratch_shapes=[pltpu.VMEM((1, h), jnp.float32)],
    )(p2[0], p2[1], hs2, d0, d1, b2.reshape(1, h), fW1,
      fb1.reshape(1, hh), fW2, fb2.reshape(1, c))

    return out
